# baseline (speedup 1.0000x reference)
import functools

import jax
import jax.numpy as jnp
from jax import lax
from jax.experimental import pallas as pl
from jax.experimental.pallas import tpu as pltpu
from jax.experimental.pallas import tpu_sc as plsc

N_NODES = 10000
N_EDGES = 320000
IN_FEATS = 128
N_HIDDEN = 128
N_CLASSES = 64

N_PAD = 10240
NC = 2
NS = 16
NW = NC * NS
CHUNK = 112
CH = 96
E_PAD = NW * CH * CHUNK
SPAN = N_PAD // NS
DUMP = N_PAD - 1
PH = 8
T_ITERS = CH // (2 * PH)
CH2 = E_PAD // (NS * CHUNK)
T2_ITERS = CH2 // (2 * PH)
BLK = 80
NBLK = SPAN // BLK

_mesh = lambda: plsc.VectorSubcoreMesh(core_axis_name="c", subcore_axis_name="s")

_SPAN_BLOCKS = []
_off = 0
while _off < SPAN:
    _SPAN_BLOCKS.append((_off, min(CHUNK, SPAN - _off)))
    _off += CHUNK


PASS_W = 64


def _hop_body(src_hbm, dst_hbm, x_hbm, u_hbm, z_hbm, *rest, do_scale):
    if do_scale:
        (y1_hbm, s_hbm,
         xsh, acc, sA, dA, sB, dB, rb0, rb1, rb2,
         g0, g1, g2, w0, w1, w2, isA, isB) = rest
    else:
        (q_hbm,
         xsh, acc, sA, dA, sB, dB, rb0, rb1, rb2,
         g0, g1, g2, w0, w1, w2, isA, isB) = rest
    c = lax.axis_index("c")
    s = lax.axis_index("s")
    base = s * SPAN
    rbufs = [rb0, rb1, rb2]
    gsems = [g0, g1, g2]
    wsems = [w0, w1, w2]

    def run_phase(idx_s, idx_d):
        gd = [None] * PH
        sd = [None] * PH
        for k in range(PH):
            b = k % 3
            if k >= 3:
                sd[k - 3].wait()
            gd[k] = pltpu.async_copy(xsh.at[idx_s.at[k]], rbufs[b], gsems[b])
            if k >= 2:
                gd[k - 2].wait()
                sd[k - 2] = pltpu.async_copy(rbufs[(k - 2) % 3],
                                             acc.at[idx_d.at[k - 2]],
                                             wsems[(k - 2) % 3], add=True)
        for k in range(PH - 2, PH):
            gd[k].wait()
            sd[k] = pltpu.async_copy(rbufs[k % 3], acc.at[idx_d.at[k]],
                                     wsems[k % 3], add=True)
        for k in range(PH - 3, PH):
            sd[k].wait()

    def hop_loop():
        def body(t, carry):
            j = t * 2 * PH
            pltpu.make_async_copy(src_hbm.at[s, pl.ds(0, PH)], sA, isA).wait()
            pltpu.make_async_copy(src_hbm.at[s, pl.ds(0, PH)], dA, isA).wait()
            pltpu.async_copy(src_hbm.at[s, pl.ds(j + PH, PH)], sB, isB)
            pltpu.async_copy(dst_hbm.at[s, pl.ds(j + PH, PH)], dB, isB)
            run_phase(sA, dA)
            pltpu.make_async_copy(src_hbm.at[s, pl.ds(0, PH)], sB, isB).wait()
            pltpu.make_async_copy(src_hbm.at[s, pl.ds(0, PH)], dB, isB).wait()

            @pl.when(t + 1 < T2_ITERS)
            def _():
                pltpu.async_copy(src_hbm.at[s, pl.ds(j + 2 * PH, PH)], sA, isA)
                pltpu.async_copy(dst_hbm.at[s, pl.ds(j + 2 * PH, PH)], dA, isA)

            run_phase(sB, dB)
            return carry

        lax.fori_loop(0, T2_ITERS, body, 0)

    def prefetch_idx():
        pltpu.async_copy(src_hbm.at[s, pl.ds(0, PH)], sA, isA)
        pltpu.async_copy(dst_hbm.at[s, pl.ds(0, PH)], dA, isA)

    def stage_blk(i, carry):
        off = base + i * BLK
        pltpu.sync_copy(x_hbm.at[c, pl.ds(off, BLK)], rb1.at[pl.ds(0, BLK)])
        pltpu.sync_copy(rb1.at[pl.ds(0, BLK)], xsh.at[pl.ds(off, BLK)])
        pltpu.sync_copy(rb0.at[pl.ds(0, BLK)], acc.at[pl.ds(off, BLK)])
        return carry

    pltpu.sync_copy(z_hbm.at[pl.ds(0, CHUNK)], rb0)
    lax.fori_loop(0, NBLK, stage_blk, 0)
    prefetch_idx()
    plsc.subcore_barrier()
    hop_loop()
    plsc.subcore_barrier()
    if do_scale:
        def scale_blk(i, carry):
            off = base + i * BLK
            pltpu.sync_copy(acc.at[pl.ds(off, BLK)], rb0.at[pl.ds(0, BLK)])
            pltpu.sync_copy(u_hbm.at[pl.ds(off, BLK)], rb1.at[pl.ds(0, BLK)])

            def scale_row(r, carry2):
                for cs in range(PASS_W // 16):
                    y = rb0[r, pl.ds(cs * 16, 16)]
                    uu = rb1[r, pl.ds(cs * 16, 16)]
                    y1v = y * uu
                    rb2[r, pl.ds(cs * 16, 16)] = y1v
                    rb0[r, pl.ds(cs * 16, 16)] = y1v * uu
                return carry2

            lax.fori_loop(0, BLK, scale_row, 0)
            pltpu.sync_copy(rb2.at[pl.ds(0, BLK)], y1_hbm.at[c, pl.ds(off, BLK)])
            pltpu.sync_copy(rb0.at[pl.ds(0, BLK)], s_hbm.at[c, pl.ds(off, BLK)])
            return carry

        lax.fori_loop(0, NBLK, scale_blk, 0)
    else:
        def out_blk(i, carry):
            off = base + i * BLK
            pltpu.sync_copy(acc.at[pl.ds(off, BLK)], rb0.at[pl.ds(0, BLK)])
            pltpu.sync_copy(rb0.at[pl.ds(0, BLK)], q_hbm.at[c, pl.ds(off, BLK)])
            return carry

        lax.fori_loop(0, NBLK, out_blk, 0)


def _deg_body(dst_hbm, const_hbm, out_hbm,
              acc, dA, dB, rb0, rb1, w0, w1, w2, w3, isA, isB):
    c = lax.axis_index("c")
    s = lax.axis_index("s")
    wid = c * NS + s
    pltpu.sync_copy(const_hbm.at[pl.ds(0, CHUNK)], rb0)
    base = s * SPAN
    for off, sz in _SPAN_BLOCKS:
        pltpu.sync_copy(rb0.at[pl.ds(0, sz)], acc.at[pl.ds(base + off, sz)])
    pltpu.async_copy(dst_hbm.at[wid, pl.ds(0, PH)], dA, isA)
    plsc.subcore_barrier()
    pltpu.sync_copy(const_hbm.at[pl.ds(CHUNK, CHUNK)], rb1)
    sems = [w0, w1, w2, w3]

    def scatter_phase(idx_d):
        sd = [None] * PH
        for k in range(PH):
            if k >= 4:
                sd[k - 4].wait()
            sd[k] = pltpu.async_copy(rb1, acc.at[idx_d.at[k]], sems[k % 4], add=True)
        for k in range(PH - 4, PH):
            sd[k].wait()

    def body(t, carry):
        j = t * 2 * PH
        pltpu.make_async_copy(dst_hbm.at[wid, pl.ds(0, PH)], dA, isA).wait()
        pltpu.async_copy(dst_hbm.at[wid, pl.ds(j + PH, PH)], dB, isB)
        scatter_phase(dA)
        pltpu.make_async_copy(dst_hbm.at[wid, pl.ds(0, PH)], dB, isB).wait()

        @pl.when(t + 1 < T_ITERS)
        def _():
            pltpu.async_copy(dst_hbm.at[wid, pl.ds(j + 2 * PH, PH)], dA, isA)

        scatter_phase(dB)
        return carry

    lax.fori_loop(0, T_ITERS, body, 0)
    plsc.subcore_barrier()
    for off, sz in _SPAN_BLOCKS:
        pltpu.sync_copy(acc.at[pl.ds(base + off, sz)], rb0.at[pl.ds(0, sz)])
        pltpu.sync_copy(rb0.at[pl.ds(0, sz)],
                        out_hbm.at[c, pl.ds(base + off, sz)])


def _sc_hop(srcr, dstr, xsplit, u64, zeros64, do_scale):
    if do_scale:
        out_type = (jax.ShapeDtypeStruct((NC, N_PAD, PASS_W), jnp.float32),
                    jax.ShapeDtypeStruct((NC, N_PAD, PASS_W), jnp.float32))
    else:
        out_type = jax.ShapeDtypeStruct((NC, N_PAD, PASS_W), jnp.float32)
    f = pl.kernel(
        functools.partial(_hop_body, do_scale=do_scale),
        out_type=out_type,
        mesh=_mesh(),
        scratch_types=[
            pltpu.VMEM_SHARED((N_PAD, PASS_W), jnp.float32),
            pltpu.VMEM_SHARED((N_PAD, PASS_W), jnp.float32),
            pltpu.VMEM((PH, CHUNK), jnp.int32),
            pltpu.VMEM((PH, CHUNK), jnp.int32),
            pltpu.VMEM((PH, CHUNK), jnp.int32),
            pltpu.VMEM((PH, CHUNK), jnp.int32),
            pltpu.VMEM((CHUNK, PASS_W), jnp.float32),
            pltpu.VMEM((CHUNK, PASS_W), jnp.float32),
            pltpu.VMEM((CHUNK, PASS_W), jnp.float32),
            pltpu.SemaphoreType.DMA,
            pltpu.SemaphoreType.DMA,
            pltpu.SemaphoreType.DMA,
            pltpu.SemaphoreType.DMA,
            pltpu.SemaphoreType.DMA,
            pltpu.SemaphoreType.DMA,
            pltpu.SemaphoreType.DMA,
            pltpu.SemaphoreType.DMA,
        ],
    )
    return f(srcr, dstr, xsplit, u64, zeros64)


def _sc_deg(dstr, consts):
    f = pl.kernel(
        _deg_body,
        out_type=jax.ShapeDtypeStruct((NC, N_PAD, PASS_W), jnp.float32),
        mesh=_mesh(),
        scratch_types=[
            pltpu.VMEM_SHARED((N_PAD, PASS_W), jnp.float32),
            pltpu.VMEM((PH, CHUNK), jnp.int32),
            pltpu.VMEM((PH, CHUNK), jnp.int32),
            pltpu.VMEM((CHUNK, PASS_W), jnp.float32),
            pltpu.VMEM((CHUNK, PASS_W), jnp.float32),
            pltpu.SemaphoreType.DMA,
            pltpu.SemaphoreType.DMA,
            pltpu.SemaphoreType.DMA,
            pltpu.SemaphoreType.DMA,
            pltpu.SemaphoreType.DMA,
            pltpu.SemaphoreType.DMA,
        ],
    )
    return f(dstr, consts)


_RA = 2048


def _norm_body(dp_ref, x_ref, u_ref, s0_ref):
    d = dp_ref[0] + dp_ref[1]
    du = 1.0 / jnp.sqrt(jnp.maximum(d, 1.0))
    u_ref[...] = du
    s = jnp.concatenate([du, du], axis=-1) * x_ref[...]
    s0_ref[0] = s[:, :PASS_W]
    s0_ref[1] = s[:, PASS_W:]


def _tc_norm(dp, x):
    grid = (N_PAD // _RA,)
    return pl.pallas_call(
        _norm_body,
        grid=grid,
        in_specs=[
            pl.BlockSpec((NC, _RA, PASS_W), lambda i: (0, i, 0)),
            pl.BlockSpec((_RA, IN_FEATS), lambda i: (i, 0)),
        ],
        out_specs=[
            pl.BlockSpec((_RA, PASS_W), lambda i: (i, 0)),
            pl.BlockSpec((2, _RA, PASS_W), lambda i: (0, i, 0)),
        ],
        out_shape=[
            jax.ShapeDtypeStruct((N_PAD, PASS_W), jnp.float32),
            jax.ShapeDtypeStruct((2, N_PAD, PASS_W), jnp.float32),
        ],
    )(dp, x)


def _out_body(q_ref, u_ref, a_ref, y1_ref, w_ref, b_ref, *out_refs, relu, with_s):
    u = u_ref[...]
    y1 = jnp.concatenate([y1_ref[0], y1_ref[1]], axis=-1)
    y2 = jnp.concatenate([u * q_ref[0], u * q_ref[1]], axis=-1)
    acc = jnp.dot(a_ref[...], w_ref[0], preferred_element_type=jnp.float32)
    acc = acc + jnp.dot(y1, w_ref[1], preferred_element_type=jnp.float32)
    acc = acc + jnp.dot(y2, w_ref[2], preferred_element_type=jnp.float32)
    h = acc + b_ref[...]
    if relu:
        h = jnp.maximum(h, 0.0)
    out_refs[0][...] = h
    if with_s:
        u2 = jnp.concatenate([u, u], axis=-1)
        sh = u2 * h
        out_refs[1][0] = sh[:, :PASS_W]
        out_refs[1][1] = sh[:, PASS_W:]


def _tc_out(q, u, a, y1, w3, b2, relu, with_s):
    grid = (N_PAD // 1024,)
    h_dim = w3.shape[-1]
    out_specs = [pl.BlockSpec((1024, h_dim), lambda i: (i, 0))]
    out_shape = [jax.ShapeDtypeStruct((N_PAD, h_dim), jnp.float32)]
    if with_s:
        out_specs.append(pl.BlockSpec((2, 1024, PASS_W), lambda i: (0, i, 0)))
        out_shape.append(jax.ShapeDtypeStruct((2, N_PAD, PASS_W), jnp.float32))
    res = pl.pallas_call(
        functools.partial(_out_body, relu=relu, with_s=with_s),
        grid=grid,
        in_specs=[
            pl.BlockSpec((NC, 1024, PASS_W), lambda i: (0, i, 0)),
            pl.BlockSpec((1024, PASS_W), lambda i: (i, 0)),
            pl.BlockSpec((1024, IN_FEATS), lambda i: (i, 0)),
            pl.BlockSpec((NC, 1024, PASS_W), lambda i: (0, i, 0)),
            pl.BlockSpec((3, IN_FEATS, h_dim), lambda i: (0, 0, 0)),
            pl.BlockSpec((1, h_dim), lambda i: (0, 0)),
        ],
        out_specs=out_specs,
        out_shape=out_shape,
    )(q, u, a, y1, w3, b2)
    return res


def kernel(features, edge_index, W0, b0, W1, b1):
    f32 = jnp.float32
    src = edge_index[0].astype(jnp.int32)
    dst = edge_index[1].astype(jnp.int32)
    pad = E_PAD - N_EDGES
    pad_dst = N_NODES + jnp.arange(pad, dtype=jnp.int32) % (N_PAD - N_NODES)
    src_flat = jnp.concatenate([src, jnp.zeros((pad,), jnp.int32)])
    dst_flat = jnp.concatenate([dst, pad_dst])
    srcr = src_flat.reshape(NS, CH2, CHUNK)
    dstr = dst_flat.reshape(NS, CH2, CHUNK)
    dstr_deg = dst_flat.reshape(NW, CH, CHUNK)
    x = jnp.pad(features.astype(f32), ((0, N_PAD - N_NODES), (0, 0)))
    consts = jnp.concatenate([jnp.zeros((CHUNK, PASS_W), f32),
                              jnp.ones((CHUNK, PASS_W), f32)])
    zeros64 = consts

    dp = _sc_deg(dstr_deg, consts)
    u, s0 = _tc_norm(dp, x)
    y1, s1 = _sc_hop(srcr, dstr, s0, u, zeros64, do_scale=True)
    q = _sc_hop(srcr, dstr, s1, u, zeros64, do_scale=False)
    h, sh = _tc_out(q, u, x, y1, W0.reshape(3, IN_FEATS, N_HIDDEN),
                    b0.reshape(1, N_HIDDEN), relu=True, with_s=True)
    y1b, s1b = _sc_hop(srcr, dstr, sh, u, zeros64, do_scale=True)
    q2 = _sc_hop(srcr, dstr, s1b, u, zeros64, do_scale=False)
    (out,) = _tc_out(q2, u, h, y1b, W1.reshape(3, N_HIDDEN, N_CLASSES),
                     b1.reshape(1, N_CLASSES), relu=False, with_s=False)
    return out[:N_NODES]


# speedup vs baseline: 1.0256x; 1.0256x over previous
import functools

import jax
import jax.numpy as jnp
from jax import lax
from jax.experimental import pallas as pl
from jax.experimental.pallas import tpu as pltpu
from jax.experimental.pallas import tpu_sc as plsc

N_NODES = 10000
N_EDGES = 320000
IN_FEATS = 128
N_HIDDEN = 128
N_CLASSES = 64

N_PAD = 10240
NC = 2
NS = 16
NW = NC * NS
CHUNK = 112
CH = 96
E_PAD = NW * CH * CHUNK
SPAN = N_PAD // NS
DUMP = N_PAD - 1
PH = 8
T_ITERS = CH // (2 * PH)
CH2 = E_PAD // (NS * CHUNK)
T2_ITERS = CH2 // (2 * PH)
BLK = 80
NBLK = SPAN // BLK

_mesh = lambda: plsc.VectorSubcoreMesh(core_axis_name="c", subcore_axis_name="s")

_SPAN_BLOCKS = []
_off = 0
while _off < SPAN:
    _SPAN_BLOCKS.append((_off, min(CHUNK, SPAN - _off)))
    _off += CHUNK


PASS_W = 64
DEG_W = 16


def _hop_body(src_hbm, dst_hbm, x_hbm, u_hbm, z_hbm, *rest, do_scale):
    if do_scale:
        (y1_hbm, s_hbm,
         xsh, acc, sA, dA, sB, dB, rb0, rb1, rb2,
         g0, g1, g2, w0, w1, w2, isA, isB) = rest
    else:
        (q_hbm,
         xsh, acc, sA, dA, sB, dB, rb0, rb1, rb2,
         g0, g1, g2, w0, w1, w2, isA, isB) = rest
    c = lax.axis_index("c")
    s = lax.axis_index("s")
    base = s * SPAN
    rbufs = [rb0, rb1, rb2]
    gsems = [g0, g1, g2]
    wsems = [w0, w1, w2]

    def run_phase(idx_s, idx_d):
        gd = [None] * PH
        sd = [None] * PH
        for k in range(PH):
            b = k % 3
            if k >= 3:
                sd[k - 3].wait()
            gd[k] = pltpu.async_copy(xsh.at[idx_s.at[k]], rbufs[b], gsems[b])
            if k >= 2:
                gd[k - 2].wait()
                sd[k - 2] = pltpu.async_copy(rbufs[(k - 2) % 3],
                                             acc.at[idx_d.at[k - 2]],
                                             wsems[(k - 2) % 3], add=True)
        for k in range(PH - 2, PH):
            gd[k].wait()
            sd[k] = pltpu.async_copy(rbufs[k % 3], acc.at[idx_d.at[k]],
                                     wsems[k % 3], add=True)
        for k in range(PH - 3, PH):
            sd[k].wait()

    def hop_loop():
        def body(t, carry):
            j = t * 2 * PH
            pltpu.make_async_copy(src_hbm.at[s, pl.ds(0, PH)], sA, isA).wait()
            pltpu.make_async_copy(src_hbm.at[s, pl.ds(0, PH)], dA, isA).wait()
            pltpu.async_copy(src_hbm.at[s, pl.ds(j + PH, PH)], sB, isB)
            pltpu.async_copy(dst_hbm.at[s, pl.ds(j + PH, PH)], dB, isB)
            run_phase(sA, dA)
            pltpu.make_async_copy(src_hbm.at[s, pl.ds(0, PH)], sB, isB).wait()
            pltpu.make_async_copy(src_hbm.at[s, pl.ds(0, PH)], dB, isB).wait()

            @pl.when(t + 1 < T2_ITERS)
            def _():
                pltpu.async_copy(src_hbm.at[s, pl.ds(j + 2 * PH, PH)], sA, isA)
                pltpu.async_copy(dst_hbm.at[s, pl.ds(j + 2 * PH, PH)], dA, isA)

            run_phase(sB, dB)
            return carry

        lax.fori_loop(0, T2_ITERS, body, 0)

    def prefetch_idx():
        pltpu.async_copy(src_hbm.at[s, pl.ds(0, PH)], sA, isA)
        pltpu.async_copy(dst_hbm.at[s, pl.ds(0, PH)], dA, isA)

    def stage_blk(i, carry):
        off = base + i * BLK
        pltpu.sync_copy(x_hbm.at[c, pl.ds(off, BLK)], rb1.at[pl.ds(0, BLK)])
        pltpu.sync_copy(rb1.at[pl.ds(0, BLK)], xsh.at[pl.ds(off, BLK)])
        pltpu.sync_copy(rb0.at[pl.ds(0, BLK)], acc.at[pl.ds(off, BLK)])
        return carry

    pltpu.sync_copy(z_hbm.at[pl.ds(0, CHUNK)], rb0)
    lax.fori_loop(0, NBLK, stage_blk, 0)
    prefetch_idx()
    plsc.subcore_barrier()
    hop_loop()
    plsc.subcore_barrier()
    if do_scale:
        def scale_blk(i, carry):
            off = base + i * BLK
            pltpu.sync_copy(acc.at[pl.ds(off, BLK)], rb0.at[pl.ds(0, BLK)])
            pltpu.sync_copy(u_hbm.at[pl.ds(off, BLK)], rb1.at[pl.ds(0, BLK)])

            def scale_row(r, carry2):
                for cs in range(PASS_W // 16):
                    y = rb0[r, pl.ds(cs * 16, 16)]
                    uu = rb1[r, pl.ds(cs * 16, 16)]
                    y1v = y * uu
                    rb2[r, pl.ds(cs * 16, 16)] = y1v
                    rb0[r, pl.ds(cs * 16, 16)] = y1v * uu
                return carry2

            lax.fori_loop(0, BLK, scale_row, 0)
            pltpu.sync_copy(rb2.at[pl.ds(0, BLK)], y1_hbm.at[c, pl.ds(off, BLK)])
            pltpu.sync_copy(rb0.at[pl.ds(0, BLK)], s_hbm.at[c, pl.ds(off, BLK)])
            return carry

        lax.fori_loop(0, NBLK, scale_blk, 0)
    else:
        def out_blk(i, carry):
            off = base + i * BLK
            pltpu.sync_copy(acc.at[pl.ds(off, BLK)], rb0.at[pl.ds(0, BLK)])
            pltpu.sync_copy(rb0.at[pl.ds(0, BLK)], q_hbm.at[c, pl.ds(off, BLK)])
            return carry

        lax.fori_loop(0, NBLK, out_blk, 0)


def _deg_body(dst_hbm, const_hbm, out_hbm,
              acc, dA, dB, rb0, rb1, w0, w1, w2, w3, isA, isB):
    c = lax.axis_index("c")
    s = lax.axis_index("s")
    wid = c * NS + s
    pltpu.sync_copy(const_hbm.at[pl.ds(0, CHUNK)], rb0)
    base = s * SPAN
    for off, sz in _SPAN_BLOCKS:
        pltpu.sync_copy(rb0.at[pl.ds(0, sz)], acc.at[pl.ds(base + off, sz)])
    pltpu.async_copy(dst_hbm.at[wid, pl.ds(0, PH)], dA, isA)
    plsc.subcore_barrier()
    pltpu.sync_copy(const_hbm.at[pl.ds(CHUNK, CHUNK)], rb1)
    sems = [w0, w1, w2, w3]

    def scatter_phase(idx_d):
        sd = [None] * PH
        for k in range(PH):
            if k >= 4:
                sd[k - 4].wait()
            sd[k] = pltpu.async_copy(rb1, acc.at[idx_d.at[k]], sems[k % 4], add=True)
        for k in range(PH - 4, PH):
            sd[k].wait()

    def body(t, carry):
        j = t * 2 * PH
        pltpu.make_async_copy(dst_hbm.at[wid, pl.ds(0, PH)], dA, isA).wait()
        pltpu.async_copy(dst_hbm.at[wid, pl.ds(j + PH, PH)], dB, isB)
        scatter_phase(dA)
        pltpu.make_async_copy(dst_hbm.at[wid, pl.ds(0, PH)], dB, isB).wait()

        @pl.when(t + 1 < T_ITERS)
        def _():
            pltpu.async_copy(dst_hbm.at[wid, pl.ds(j + 2 * PH, PH)], dA, isA)

        scatter_phase(dB)
        return carry

    lax.fori_loop(0, T_ITERS, body, 0)
    plsc.subcore_barrier()
    for off, sz in _SPAN_BLOCKS:
        pltpu.sync_copy(acc.at[pl.ds(base + off, sz)], rb0.at[pl.ds(0, sz)])
        pltpu.sync_copy(rb0.at[pl.ds(0, sz)],
                        out_hbm.at[c, pl.ds(base + off, sz)])


def _sc_hop(srcr, dstr, xsplit, u64, zeros64, do_scale):
    if do_scale:
        out_type = (jax.ShapeDtypeStruct((NC, N_PAD, PASS_W), jnp.float32),
                    jax.ShapeDtypeStruct((NC, N_PAD, PASS_W), jnp.float32))
    else:
        out_type = jax.ShapeDtypeStruct((NC, N_PAD, PASS_W), jnp.float32)
    f = pl.kernel(
        functools.partial(_hop_body, do_scale=do_scale),
        out_type=out_type,
        mesh=_mesh(),
        scratch_types=[
            pltpu.VMEM_SHARED((N_PAD, PASS_W), jnp.float32),
            pltpu.VMEM_SHARED((N_PAD, PASS_W), jnp.float32),
            pltpu.VMEM((PH, CHUNK), jnp.int32),
            pltpu.VMEM((PH, CHUNK), jnp.int32),
            pltpu.VMEM((PH, CHUNK), jnp.int32),
            pltpu.VMEM((PH, CHUNK), jnp.int32),
            pltpu.VMEM((CHUNK, PASS_W), jnp.float32),
            pltpu.VMEM((CHUNK, PASS_W), jnp.float32),
            pltpu.VMEM((CHUNK, PASS_W), jnp.float32),
            pltpu.SemaphoreType.DMA,
            pltpu.SemaphoreType.DMA,
            pltpu.SemaphoreType.DMA,
            pltpu.SemaphoreType.DMA,
            pltpu.SemaphoreType.DMA,
            pltpu.SemaphoreType.DMA,
            pltpu.SemaphoreType.DMA,
            pltpu.SemaphoreType.DMA,
        ],
    )
    return f(srcr, dstr, xsplit, u64, zeros64)


def _sc_deg(dstr, consts):
    f = pl.kernel(
        _deg_body,
        out_type=jax.ShapeDtypeStruct((NC, N_PAD, DEG_W), jnp.float32),
        mesh=_mesh(),
        scratch_types=[
            pltpu.VMEM_SHARED((N_PAD, DEG_W), jnp.float32),
            pltpu.VMEM((PH, CHUNK), jnp.int32),
            pltpu.VMEM((PH, CHUNK), jnp.int32),
            pltpu.VMEM((CHUNK, DEG_W), jnp.float32),
            pltpu.VMEM((CHUNK, DEG_W), jnp.float32),
            pltpu.SemaphoreType.DMA,
            pltpu.SemaphoreType.DMA,
            pltpu.SemaphoreType.DMA,
            pltpu.SemaphoreType.DMA,
            pltpu.SemaphoreType.DMA,
            pltpu.SemaphoreType.DMA,
        ],
    )
    return f(dstr, consts)


_RA = 2048


def _norm_body(dp_ref, x_ref, u_ref, s0_ref):
    d = dp_ref[0] + dp_ref[1]
    du16 = 1.0 / jnp.sqrt(jnp.maximum(d, 1.0))
    du = jnp.concatenate([du16] * (PASS_W // DEG_W), axis=-1)
    u_ref[...] = du
    s = jnp.concatenate([du, du], axis=-1) * x_ref[...]
    s0_ref[0] = s[:, :PASS_W]
    s0_ref[1] = s[:, PASS_W:]


def _tc_norm(dp, x):
    grid = (N_PAD // _RA,)
    return pl.pallas_call(
        _norm_body,
        grid=grid,
        in_specs=[
            pl.BlockSpec((NC, _RA, DEG_W), lambda i: (0, i, 0)),
            pl.BlockSpec((_RA, IN_FEATS), lambda i: (i, 0)),
        ],
        out_specs=[
            pl.BlockSpec((_RA, PASS_W), lambda i: (i, 0)),
            pl.BlockSpec((2, _RA, PASS_W), lambda i: (0, i, 0)),
        ],
        out_shape=[
            jax.ShapeDtypeStruct((N_PAD, PASS_W), jnp.float32),
            jax.ShapeDtypeStruct((2, N_PAD, PASS_W), jnp.float32),
        ],
    )(dp, x)


def _out_body(q_ref, u_ref, a_ref, y1_ref, w_ref, b_ref, *out_refs, relu, with_s):
    u = u_ref[...]
    y1 = jnp.concatenate([y1_ref[0], y1_ref[1]], axis=-1)
    y2 = jnp.concatenate([u * q_ref[0], u * q_ref[1]], axis=-1)
    acc = jnp.dot(a_ref[...], w_ref[0], preferred_element_type=jnp.float32)
    acc = acc + jnp.dot(y1, w_ref[1], preferred_element_type=jnp.float32)
    acc = acc + jnp.dot(y2, w_ref[2], preferred_element_type=jnp.float32)
    h = acc + b_ref[...]
    if relu:
        h = jnp.maximum(h, 0.0)
    out_refs[0][...] = h
    if with_s:
        u2 = jnp.concatenate([u, u], axis=-1)
        sh = u2 * h
        out_refs[1][0] = sh[:, :PASS_W]
        out_refs[1][1] = sh[:, PASS_W:]


def _tc_out(q, u, a, y1, w3, b2, relu, with_s):
    grid = (N_PAD // 1024,)
    h_dim = w3.shape[-1]
    out_specs = [pl.BlockSpec((1024, h_dim), lambda i: (i, 0))]
    out_shape = [jax.ShapeDtypeStruct((N_PAD, h_dim), jnp.float32)]
    if with_s:
        out_specs.append(pl.BlockSpec((2, 1024, PASS_W), lambda i: (0, i, 0)))
        out_shape.append(jax.ShapeDtypeStruct((2, N_PAD, PASS_W), jnp.float32))
    res = pl.pallas_call(
        functools.partial(_out_body, relu=relu, with_s=with_s),
        grid=grid,
        in_specs=[
            pl.BlockSpec((NC, 1024, PASS_W), lambda i: (0, i, 0)),
            pl.BlockSpec((1024, PASS_W), lambda i: (i, 0)),
            pl.BlockSpec((1024, IN_FEATS), lambda i: (i, 0)),
            pl.BlockSpec((NC, 1024, PASS_W), lambda i: (0, i, 0)),
            pl.BlockSpec((3, IN_FEATS, h_dim), lambda i: (0, 0, 0)),
            pl.BlockSpec((1, h_dim), lambda i: (0, 0)),
        ],
        out_specs=out_specs,
        out_shape=out_shape,
    )(q, u, a, y1, w3, b2)
    return res


def kernel(features, edge_index, W0, b0, W1, b1):
    f32 = jnp.float32
    src = edge_index[0].astype(jnp.int32)
    dst = edge_index[1].astype(jnp.int32)
    pad = E_PAD - N_EDGES
    pad_dst = N_NODES + jnp.arange(pad, dtype=jnp.int32) % (N_PAD - N_NODES)
    src_flat = jnp.concatenate([src, jnp.zeros((pad,), jnp.int32)])
    dst_flat = jnp.concatenate([dst, pad_dst])
    srcr = src_flat.reshape(NS, CH2, CHUNK)
    dstr = dst_flat.reshape(NS, CH2, CHUNK)
    dstr_deg = dst_flat.reshape(NW, CH, CHUNK)
    x = jnp.pad(features.astype(f32), ((0, N_PAD - N_NODES), (0, 0)))
    consts16 = jnp.concatenate([jnp.zeros((CHUNK, DEG_W), f32),
                                jnp.ones((CHUNK, DEG_W), f32)])
    zeros64 = jnp.zeros((CHUNK, PASS_W), f32)

    dp = _sc_deg(dstr_deg, consts16)
    u, s0 = _tc_norm(dp, x)
    y1, s1 = _sc_hop(srcr, dstr, s0, u, zeros64, do_scale=True)
    q = _sc_hop(srcr, dstr, s1, u, zeros64, do_scale=False)
    h, sh = _tc_out(q, u, x, y1, W0.reshape(3, IN_FEATS, N_HIDDEN),
                    b0.reshape(1, N_HIDDEN), relu=True, with_s=True)
    y1b, s1b = _sc_hop(srcr, dstr, sh, u, zeros64, do_scale=True)
    q2 = _sc_hop(srcr, dstr, s1b, u, zeros64, do_scale=False)
    (out,) = _tc_out(q2, u, h, y1b, W1.reshape(3, N_HIDDEN, N_CLASSES),
                     b1.reshape(1, N_CLASSES), relu=False, with_s=False)
    return out[:N_NODES]
